# baseline (speedup 1.0000x reference)
import functools

import jax
import jax.numpy as jnp
from jax import lax
from jax.experimental import pallas as pl
from jax.experimental.pallas import tpu as pltpu
from jax.experimental.pallas import tpu_sc as plsc

_D = 32
_N_CODE = 8192
_N_TOK = 8192
_T = 256
_NW = 32
_BPW = _N_TOK // _NW


_C = 512
_NJ = _N_CODE // _C


_GRP = 4


def _argmin_body(x_ref, e_ref, idx_ref, dsum_ref,
                 gmin_ref, garg_ref, amin_ref, aarg_ref, aval_ref):
    i = pl.program_id(0)
    j = pl.program_id(1)

    @pl.when(j == 0)
    def _():
        amin_ref[...] = jnp.full((1, _T), jnp.inf, jnp.float32)
        aarg_ref[...] = jnp.zeros((1, _T), jnp.int32)
        aval_ref[...] = jnp.full((1, _T), jnp.inf, jnp.float32)

    xb = x_ref[...]
    ec = e_ref[...]
    scores = lax.dot_general(
        ec.astype(jnp.bfloat16), xb.astype(jnp.bfloat16),
        (((1,), (1,)), ((), ())),
        preferred_element_type=jnp.float32)
    esq = jnp.sum(ec * ec, axis=1, keepdims=True)
    sq = xb * xb
    xsq = lax.dot_general(
        jnp.ones((1, _D), jnp.float32), sq,
        (((1,), (1,)), ((), ())),
        preferred_element_type=jnp.float32,
        precision=lax.Precision.HIGHEST)
    d = (xsq + esq) - 2.0 * scores
    cmin = jnp.min(d, axis=0)[None, :]
    rows = lax.broadcasted_iota(jnp.int32, d.shape, 0) + j * _C
    carg = jnp.min(jnp.where(d == cmin, rows, _N_CODE), axis=0)[None, :]

    @pl.when(j % _GRP == 0)
    def _():
        gmin_ref[...] = cmin
        garg_ref[...] = carg

    @pl.when(j % _GRP != 0)
    def _():
        better = cmin < gmin_ref[...]
        garg_ref[...] = jnp.where(better, carg, garg_ref[...])
        gmin_ref[...] = jnp.where(better, cmin, gmin_ref[...])

    @pl.when(j % _GRP == _GRP - 1)
    def _():
        upd = gmin_ref[...] < amin_ref[...]
        aarg_ref[...] = jnp.where(upd, garg_ref[...], aarg_ref[...])
        merged = jnp.where(upd, gmin_ref[...], amin_ref[...])
        amin_ref[...] = merged.astype(jnp.bfloat16).astype(jnp.float32)
        aval_ref[...] = jnp.where(upd, gmin_ref[...], aval_ref[...])

    @pl.when(j == _NJ - 1)
    def _():
        idx_ref[...] = aarg_ref[...].reshape(1, 1, _T)
        partial = jnp.sum(aval_ref[...])

        @pl.when(i == 0)
        def _():
            dsum_ref[0, 0] = 0.0

        dsum_ref[0, 0] += partial


_argmin_call = pl.pallas_call(
    _argmin_body,
    grid=(_N_TOK // _T, _NJ),
    in_specs=[
        pl.BlockSpec((_T, _D), lambda i, j: (i, 0)),
        pl.BlockSpec((_C, _D), lambda i, j: (j, 0)),
    ],
    out_specs=[
        pl.BlockSpec((1, 1, _T), lambda i, j: (i, 0, 0)),
        pl.BlockSpec(memory_space=pltpu.SMEM, block_shape=(1, 1),
                     index_map=lambda i, j: (0, 0)),
    ],
    out_shape=[
        jax.ShapeDtypeStruct((_N_TOK // _T, 1, _T), jnp.int32),
        jax.ShapeDtypeStruct((1, 1), jnp.float32),
    ],
    scratch_shapes=[
        pltpu.VMEM((1, _T), jnp.float32),
        pltpu.VMEM((1, _T), jnp.int32),
        pltpu.VMEM((1, _T), jnp.float32),
        pltpu.VMEM((1, _T), jnp.int32),
        pltpu.VMEM((1, _T), jnp.float32),
    ],
)


def _gather_body(table_hbm, idx_hbm, out_hbm, idx_v, rows_v, sem):
    wid = lax.axis_index("s") * 2 + lax.axis_index("c")
    base = wid * _BPW
    pltpu.sync_copy(idx_hbm.at[pl.ds(base, _BPW)], idx_v)
    pltpu.async_copy(table_hbm.at[idx_v], rows_v, sem).wait()
    pltpu.sync_copy(rows_v, out_hbm.at[pl.ds(base, _BPW)])


@functools.lru_cache(maxsize=1)
def _make_gather_call():
    return functools.partial(
        pl.kernel,
        mesh=plsc.VectorSubcoreMesh(core_axis_name="c", subcore_axis_name="s"),
        out_type=jax.ShapeDtypeStruct((_N_TOK, _D), jnp.float32),
        scratch_types=[
            pltpu.VMEM((_BPW,), jnp.int32),
            pltpu.VMEM((_BPW, _D), jnp.float32),
            pltpu.SemaphoreType.DMA,
        ],
        compiler_params=pltpu.CompilerParams(use_tc_tiling_on_sc=False),
    )(_gather_body)


@jax.jit
def kernel(x, embeddings):
    flat_x = x.reshape(-1, _D)
    idx2d, dsum = _argmin_call(flat_x, embeddings)
    idx = idx2d.reshape(_N_TOK)
    flat_q = _make_gather_call()(embeddings, idx)
    quantized = flat_q.reshape(x.shape)
    loss = (1.0 + 0.25) * dsum[0, 0] / jnp.float32(x.size)
    quantized_st = x + (quantized - x)
    return (quantized_st, loss)


# speedup vs baseline: 1.0862x; 1.0862x over previous
import functools

import jax
import jax.numpy as jnp
from jax import lax
from jax.experimental import pallas as pl
from jax.experimental.pallas import tpu as pltpu
from jax.experimental.pallas import tpu_sc as plsc

_D = 32
_N_CODE = 8192
_N_TOK = 8192
_T = 256
_NW = 32
_BPW = _N_TOK // _NW


_C = 512
_NJ = _N_CODE // _C


_GRP = 4


def _argmin_body(x_ref, e_ref, idx_ref, dsum_ref,
                 gmin_ref, garg_ref, amin_ref, aarg_ref, aval_ref,
                 ebf_ref, esq_ref, xbf_ref, xsq_ref):
    i = pl.program_id(0)
    j = pl.program_id(1)

    @pl.when(i == 0)
    def _():
        ec = e_ref[...]
        ebf_ref[pl.ds(j * _C, _C), :] = (ec * -2.0).astype(jnp.bfloat16)
        esq_ref[pl.ds(j * _C, _C), :] = jnp.sum(ec * ec, axis=1,
                                                keepdims=True)

    @pl.when(j == 0)
    def _():
        xb = x_ref[...]
        xbf_ref[...] = xb.astype(jnp.bfloat16)
        xsq_ref[...] = lax.dot_general(
            jnp.ones((1, _D), jnp.float32), xb * xb,
            (((1,), (1,)), ((), ())),
            preferred_element_type=jnp.float32,
            precision=lax.Precision.HIGHEST)
        amin_ref[...] = jnp.full((1, _T), jnp.inf, jnp.float32)
        aarg_ref[...] = jnp.zeros((1, _T), jnp.int32)
        aval_ref[...] = jnp.full((1, _T), jnp.inf, jnp.float32)

    sneg2 = lax.dot_general(
        ebf_ref[pl.ds(j * _C, _C), :], xbf_ref[...],
        (((1,), (1,)), ((), ())),
        preferred_element_type=jnp.float32)
    d = (xsq_ref[...] + esq_ref[pl.ds(j * _C, _C), :]) + sneg2
    cmin = jnp.min(d, axis=0)[None, :]
    rows = lax.broadcasted_iota(jnp.int32, d.shape, 0)
    carg = jnp.min(jnp.where(d == cmin, rows, _N_CODE),
                   axis=0)[None, :] + j * _C

    @pl.when(j % _GRP == 0)
    def _():
        gmin_ref[...] = cmin
        garg_ref[...] = carg

    @pl.when(j % _GRP != 0)
    def _():
        better = cmin < gmin_ref[...]
        garg_ref[...] = jnp.where(better, carg, garg_ref[...])
        gmin_ref[...] = jnp.where(better, cmin, gmin_ref[...])

    @pl.when(j % _GRP == _GRP - 1)
    def _():
        upd = gmin_ref[...] < amin_ref[...]
        aarg_ref[...] = jnp.where(upd, garg_ref[...], aarg_ref[...])
        merged = jnp.where(upd, gmin_ref[...], amin_ref[...])
        amin_ref[...] = merged.astype(jnp.bfloat16).astype(jnp.float32)
        aval_ref[...] = jnp.where(upd, gmin_ref[...], aval_ref[...])

    @pl.when(j == _NJ - 1)
    def _():
        idx_ref[...] = aarg_ref[...].reshape(1, 1, _T)
        partial = jnp.sum(aval_ref[...])

        @pl.when(i == 0)
        def _():
            dsum_ref[0, 0] = 0.0

        dsum_ref[0, 0] += partial


_argmin_call = pl.pallas_call(
    _argmin_body,
    grid=(_N_TOK // _T, _NJ),
    in_specs=[
        pl.BlockSpec((_T, _D), lambda i, j: (i, 0)),
        pl.BlockSpec((_C, _D), lambda i, j: (j, 0)),
    ],
    out_specs=[
        pl.BlockSpec((1, 1, _T), lambda i, j: (i, 0, 0)),
        pl.BlockSpec(memory_space=pltpu.SMEM, block_shape=(1, 1),
                     index_map=lambda i, j: (0, 0)),
    ],
    out_shape=[
        jax.ShapeDtypeStruct((_N_TOK // _T, 1, _T), jnp.int32),
        jax.ShapeDtypeStruct((1, 1), jnp.float32),
    ],
    scratch_shapes=[
        pltpu.VMEM((1, _T), jnp.float32),
        pltpu.VMEM((1, _T), jnp.int32),
        pltpu.VMEM((1, _T), jnp.float32),
        pltpu.VMEM((1, _T), jnp.int32),
        pltpu.VMEM((1, _T), jnp.float32),
        pltpu.VMEM((_N_CODE, _D), jnp.bfloat16),
        pltpu.VMEM((_N_CODE, 1), jnp.float32),
        pltpu.VMEM((_T, _D), jnp.bfloat16),
        pltpu.VMEM((1, _T), jnp.float32),
    ],
)


def _gather_body(table_hbm, idx_hbm, out_hbm, idx_v, rows_v, sem):
    wid = lax.axis_index("s") * 2 + lax.axis_index("c")
    base = wid * _BPW
    pltpu.sync_copy(idx_hbm.at[pl.ds(base, _BPW)], idx_v)
    pltpu.async_copy(table_hbm.at[idx_v], rows_v, sem).wait()
    pltpu.sync_copy(rows_v, out_hbm.at[pl.ds(base, _BPW)])


@functools.lru_cache(maxsize=1)
def _make_gather_call():
    return functools.partial(
        pl.kernel,
        mesh=plsc.VectorSubcoreMesh(core_axis_name="c", subcore_axis_name="s"),
        out_type=jax.ShapeDtypeStruct((_N_TOK, _D), jnp.float32),
        scratch_types=[
            pltpu.VMEM((_BPW,), jnp.int32),
            pltpu.VMEM((_BPW, _D), jnp.float32),
            pltpu.SemaphoreType.DMA,
        ],
        compiler_params=pltpu.CompilerParams(use_tc_tiling_on_sc=False),
    )(_gather_body)


@jax.jit
def kernel(x, embeddings):
    flat_x = x.reshape(-1, _D)
    idx2d, dsum = _argmin_call(flat_x, embeddings)
    idx = idx2d.reshape(_N_TOK)
    flat_q = _make_gather_call()(embeddings, idx)
    quantized = flat_q.reshape(x.shape)
    loss = (1.0 + 0.25) * dsum[0, 0] / jnp.float32(x.size)
    quantized_st = x + (quantized - x)
    return (quantized_st, loss)
